# baseline (speedup 1.0000x reference)
import numpy as np
import jax
import jax.numpy as jnp
from jax import lax
from jax.experimental import pallas as pl
from jax.experimental.pallas import tpu as pltpu

N = 8192
IN = 64
SD = 4
PD = 22
OUT = 96
K = 40
NEV = 8
TILE = 512
C = 512
BIG = 1e9
KEY_HI = int(np.asarray(BIG, np.float32).view(np.int32))


def _proj_body(x_ref, Ws_ref, bs_ref, Wh_ref, bh_ref, s_ref, sn_ref, h_ref):
    x = x_ref[...]
    s = jnp.dot(x, Ws_ref[...], preferred_element_type=jnp.float32) + bs_ref[...]
    h = jnp.dot(x, Wh_ref[...], preferred_element_type=jnp.float32) + bh_ref[...]
    s_ref[...] = s
    sn_ref[...] = jnp.sum(s * s, axis=1, keepdims=True)
    h_ref[...] = h


def _grav_body(s_ref, sn_ref, bc_ref, hT_ref, h_ref, meta_ref, sTt_ref,
               snt_ref, brt_ref, x_ref, Wo1_ref, Wo2a_ref, Wo2b_ref, bo2_ref,
               xg_ref, dsT_ref, wb_ref):
    wlo = meta_ref[0, 0, 0]
    nch = meta_ref[0, 0, 1]
    sTt = sTt_ref[...]
    snt = snt_ref[...]
    brt = brt_ref[...]

    def fill(c, _):
        off = pl.multiple_of(wlo + c * C, C)
        G = jnp.dot(s_ref[pl.ds(off, C), :], sTt,
                    preferred_element_type=jnp.float32)
        d2 = (sn_ref[pl.ds(off, C), :] + snt) - 2.0 * G
        d2 = jnp.where(bc_ref[pl.ds(off, C), :] != brt, BIG, d2)
        dsT_ref[pl.ds(off, C), :] = d2
        return 0

    lax.fori_loop(0, nch, fill, 0)

    lo0 = jnp.zeros((1, TILE), jnp.int32)
    hi0 = jnp.full((1, TILE), KEY_HI, jnp.int32)

    def bisect(_, carry):
        lo, hi = carry
        mid = lo + (hi - lo) // 2
        midf = lax.bitcast_convert_type(mid, jnp.float32)

        def cpart(c, acc):
            off = pl.multiple_of(wlo + c * C, C)
            v = dsT_ref[pl.ds(off, C), :]
            return acc + jnp.sum((v <= midf).astype(jnp.int32), axis=0,
                                 keepdims=True)

        cnt = lax.fori_loop(0, nch, cpart, jnp.zeros((1, TILE), jnp.int32))
        ge = cnt >= K
        return jnp.where(ge, lo, mid + 1), jnp.where(ge, mid, hi)

    _, hi = lax.fori_loop(0, 31, bisect, (lo0, hi0))
    tf = lax.bitcast_convert_type(hi, jnp.float32)

    def mpart(c, acc):
        off = pl.multiple_of(wlo + c * C, C)
        v = dsT_ref[pl.ds(off, C), :]
        sel = v <= tf
        wv = jnp.where(sel, jnp.maximum(jnp.exp(-10.0 * v), 1e-30), 0.0)
        wb_ref[pl.ds(off, C), :] = wv
        return acc + jnp.dot(hT_ref[:, pl.ds(off, C)], wv,
                             preferred_element_type=jnp.float32,
                             precision=lax.Precision.HIGHEST)

    meanT = lax.fori_loop(0, nch, mpart,
                          jnp.zeros((PD, TILE), jnp.float32)) * (1.0 / K)

    rows = []
    for f in range(PD):
        def xpart(c, acc, f=f):
            off = pl.multiple_of(wlo + c * C, C)
            wv = wb_ref[pl.ds(off, C), :]
            v = jnp.where(wv > 0.0, wv * h_ref[pl.ds(off, C), f:f + 1], -1e30)
            return jnp.maximum(acc, jnp.max(v, axis=0, keepdims=True))

        rows.append(lax.fori_loop(0, nch, xpart,
                                  jnp.full((1, TILE), -3e38, jnp.float32)))
    maxT = jnp.concatenate(rows, axis=0)

    tdims = (((0,), (0,)), ((), ()))
    xg = (jnp.dot(x_ref[...], Wo1_ref[...], preferred_element_type=jnp.float32)
          + lax.dot_general(meanT, Wo2a_ref[...], tdims,
                            preferred_element_type=jnp.float32)
          + lax.dot_general(maxT, Wo2b_ref[...], tdims,
                            preferred_element_type=jnp.float32)
          + bo2_ref[...])
    xg_ref[...] = xg


def _colstats_body(x_ref, s_ref, q_ref):
    xv = x_ref[...]
    s_ref[...] = jnp.sum(xv, axis=0, keepdims=True)
    q_ref[...] = jnp.sum(xv * xv, axis=0, keepdims=True)


def _bn_from_stats(xv, s, q, g, b, eps=1e-5):
    m = s * (1.0 / N)
    v = q * (1.0 / N) - m * m
    return (xv - m) / jnp.sqrt(v + eps) * g + b


def _bnlin_body(x_ref, s_ref, q_ref, g_ref, b_ref, W_ref, bias_ref, o_ref):
    xn = _bn_from_stats(x_ref[...], s_ref[...], q_ref[...], g_ref[...], b_ref[...])
    o_ref[...] = jnp.tanh(
        jnp.dot(xn, W_ref[...], preferred_element_type=jnp.float32) + bias_ref[...])


def _seg_body(y_ref, bc_ref, stats_ref, cnt_ref):
    e = pl.program_id(0)
    y = y_ref[...]
    mask = bc_ref[...] == e
    mf = mask.astype(jnp.float32)
    ssum = jnp.sum(y * mf, axis=0, keepdims=True)
    smin = jnp.min(jnp.where(mask, y, 1e30), axis=0, keepdims=True)
    smax = jnp.max(jnp.where(mask, y, -1e30), axis=0, keepdims=True)
    stats_ref[...] = jnp.concatenate([ssum, smin, smax], axis=1).reshape(
        1, 1, 3 * OUT)
    cnt_ref[...] = jnp.reshape(jnp.sum(mf), (1, 1, 1))


def _final_body(y_ref, bc_ref, stats_ref, cnt_ref, WoutA_ref, WoutB_ref,
                bout_ref, xo_ref):
    st = stats_ref[...]
    cnt = jnp.maximum(cnt_ref[...], 1.0)
    seg = jnp.concatenate([st[:, :OUT] / cnt, st[:, OUT:]], axis=1)
    s2 = jnp.dot(seg, WoutA_ref[...], preferred_element_type=jnp.float32)
    oh = (bc_ref[...] == lax.broadcasted_iota(jnp.int32, (N, NEV), 1)
          ).astype(jnp.float32)
    contrib = jnp.dot(oh, s2, preferred_element_type=jnp.float32, precision=lax.Precision.HIGHEST)
    xo_ref[...] = jnp.tanh(
        jnp.dot(y_ref[...], WoutB_ref[...], preferred_element_type=jnp.float32)
        + contrib + bout_ref[...])


def _bn3_body(x_ref, s_ref, q_ref, g_ref, b_ref, o_ref):
    o_ref[...] = _bn_from_stats(x_ref[...], s_ref[...], q_ref[...],
                                g_ref[...], b_ref[...])


def _full(shape):
    nd = len(shape)
    return pl.BlockSpec(shape, lambda i: (0,) * nd)


def _colstats(xv, c):
    return pl.pallas_call(
        _colstats_body,
        out_shape=[jax.ShapeDtypeStruct((1, c), jnp.float32),
                   jax.ShapeDtypeStruct((1, c), jnp.float32)],
    )(xv)


def kernel(x, batch, Ws, bs, Wh, bh, Wo1, Wo2, bo2, bn1g, bn1b, W1, b1,
           bn2g, bn2b, W2, b2, Wout, bout, bn3g, bn3b):
    batch = batch.astype(jnp.int32)
    r = lambda v: v.reshape(1, -1)

    s, sn, h = pl.pallas_call(
        _proj_body,
        out_shape=[jax.ShapeDtypeStruct((N, SD), jnp.float32),
                   jax.ShapeDtypeStruct((N, 1), jnp.float32),
                   jax.ShapeDtypeStruct((N, PD), jnp.float32)],
    )(x, Ws, r(bs), Wh, r(bh))

    sT = s.T
    snrow = sn.reshape(1, N)
    hT = h.T
    ntiles = N // TILE
    firsts = batch[::TILE]
    lasts = batch[TILE - 1::TILE]
    lo_i = jnp.searchsorted(batch, firsts, side="left").astype(jnp.int32)
    hi_i = jnp.searchsorted(batch, lasts, side="right").astype(jnp.int32)
    wlo = (lo_i // C) * C
    nch = (-(-(hi_i - wlo) // C)).astype(jnp.int32)
    meta = jnp.stack([wlo, nch], axis=1).reshape(ntiles, 1, 2)
    br = batch.reshape(1, N)
    bc = batch.reshape(N, 1)

    xg = pl.pallas_call(
        _grav_body,
        grid=(ntiles,),
        in_specs=[
            _full((N, SD)),
            _full((N, 1)),
            _full((N, 1)),
            _full((PD, N)),
            _full((N, PD)),
            pl.BlockSpec((1, 1, 2), lambda i: (i, 0, 0),
                         memory_space=pltpu.SMEM),
            pl.BlockSpec((SD, TILE), lambda i: (0, i)),
            pl.BlockSpec((1, TILE), lambda i: (0, i)),
            pl.BlockSpec((1, TILE), lambda i: (0, i)),
            pl.BlockSpec((TILE, IN), lambda i: (i, 0)),
            _full((IN, OUT)),
            _full((PD, OUT)),
            _full((PD, OUT)),
            _full((1, OUT)),
        ],
        out_specs=pl.BlockSpec((TILE, OUT), lambda i: (i, 0)),
        out_shape=jax.ShapeDtypeStruct((N, OUT), jnp.float32),
        scratch_shapes=[pltpu.VMEM((N, TILE), jnp.float32),
                        pltpu.VMEM((N, TILE), jnp.float32)],
        compiler_params=pltpu.CompilerParams(
            dimension_semantics=("arbitrary",)),
    )(s, sn, bc, hT, h, meta, sT, snrow, br, x, Wo1, Wo2[:PD], Wo2[PD:],
      r(bo2))

    s1, q1 = _colstats(xg, OUT)
    y1 = pl.pallas_call(
        _bnlin_body,
        out_shape=jax.ShapeDtypeStruct((N, 128), jnp.float32),
    )(xg, s1, q1, r(bn1g), r(bn1b), W1, r(b1))

    s2_, q2_ = _colstats(y1, 128)
    y2 = pl.pallas_call(
        _bnlin_body,
        out_shape=jax.ShapeDtypeStruct((N, OUT), jnp.float32),
    )(y1, s2_, q2_, r(bn2g), r(bn2b), W2, r(b2))

    stats, cnt = pl.pallas_call(
        _seg_body,
        grid=(NEV,),
        in_specs=[_full((N, OUT)), _full((N, 1))],
        out_specs=[pl.BlockSpec((1, 1, 3 * OUT), lambda e: (e, 0, 0)),
                   pl.BlockSpec((1, 1, 1), lambda e: (e, 0, 0))],
        out_shape=[jax.ShapeDtypeStruct((NEV, 1, 3 * OUT), jnp.float32),
                   jax.ShapeDtypeStruct((NEV, 1, 1), jnp.float32)],
        compiler_params=pltpu.CompilerParams(
            dimension_semantics=("arbitrary",)),
    )(y2, bc)
    stats = stats.reshape(NEV, 3 * OUT)
    cnt = cnt.reshape(NEV, 1)

    xo = pl.pallas_call(
        _final_body,
        out_shape=jax.ShapeDtypeStruct((N, OUT), jnp.float32),
    )(y2, bc, stats, cnt, Wout[:3 * OUT], Wout[3 * OUT:], r(bout))

    s3, q3 = _colstats(xo, OUT)
    out = pl.pallas_call(
        _bn3_body,
        out_shape=jax.ShapeDtypeStruct((N, OUT), jnp.float32),
    )(xo, s3, q3, r(bn3g), r(bn3b))
    return out
